# Initial kernel scaffold; baseline (speedup 1.0000x reference)
import jax
import jax.numpy as jnp
from jax.experimental import pallas as pl


def kernel(x, lin_W, lin_b, bias0, comp1, basis1, bias1, comp2, basis2, bias2, edge_index, edge_type):
    raise NotImplementedError("write your pallas kernel here")


# speedup vs baseline: 4.8542x; 4.8542x over previous
import functools

import jax
import jax.numpy as jnp
from jax import lax
from jax.experimental import pallas as pl
from jax.experimental.pallas import tpu as pltpu
from jax.experimental.pallas import tpu_sc as plsc

N = 10000
E = 160000
R = 4
IN_DIM = 128
H_DIM = 256
OUT_DIM = 64
NUM_B = 2

CW = 32
NCHUNK = H_DIM // CW
CORES = 2
TILES = 16
RN = R * N
SLAB_ROWS = 40960
STRIPE = SLAB_ROWS // TILES
DUMMY_ROW = RN
BATCH = 128
EPT = 10240
NBATCH = EPT // BATCH
E_PAD = EPT * TILES


def _sc_body(table, src2, dst2, et2, agg_out, slab, srcv, flatv, adjv,
             rowbuf, zbuf):
    core = lax.axis_index("c")
    sub = lax.axis_index("s")

    pltpu.sync_copy(src2.at[pl.ds(sub * NBATCH, NBATCH)], srcv)
    pltpu.sync_copy(dst2.at[pl.ds(sub * NBATCH, NBATCH)], flatv)
    pltpu.sync_copy(et2.at[pl.ds(sub * NBATCH, NBATCH)], adjv)

    def _flat_body(j, carry):
        for k in range(BATCH // 16):
            sl = pl.ds(k * 16, 16)
            flatv[j, sl] = adjv[j, sl] * N + flatv[j, sl]
        return carry
    lax.fori_loop(0, NBATCH, _flat_body, 0)

    def _zb(j, carry):
        z = jnp.zeros((16,), jnp.float32)
        for k in range(CW // 16):
            zbuf[j, pl.ds(k * 16, 16)] = z
        return carry
    lax.fori_loop(0, BATCH, _zb, 0)

    for lc in range(NCHUNK // CORES):
        chunk = core * (NCHUNK // CORES) + lc

        def _adj(j, carry):
            for k in range(BATCH // 16):
                sl = pl.ds(k * 16, 16)
                adjv[j, sl] = srcv[j, sl] * NCHUNK + chunk
            return carry
        lax.fori_loop(0, NBATCH, _adj, 0)

        def _zero(t, carry):
            pltpu.sync_copy(zbuf, slab.at[pl.ds(sub * STRIPE + t * BATCH, BATCH)])
            return carry
        lax.fori_loop(0, STRIPE // BATCH, _zero, 0)
        plsc.subcore_barrier()

        def _edge(b, carry):
            pltpu.sync_copy(table.at[adjv.at[b]], rowbuf)
            pltpu.sync_copy(rowbuf, slab.at[flatv.at[b]], add=True)
            return carry
        lax.fori_loop(0, NBATCH, _edge, 0)
        plsc.subcore_barrier()

        pltpu.sync_copy(
            slab.at[pl.ds(sub * STRIPE, STRIPE)],
            agg_out.at[pl.ds(sub * STRIPE, STRIPE), pl.ds(chunk * CW, CW)])


def _sc_deg_body(dst2, et2, deg_out, deg_slab, flatv, etv, onesv, z16):
    core = lax.axis_index("c")
    sub = lax.axis_index("s")

    @pl.when(core == 0)
    def _deg_pass():
        pltpu.sync_copy(dst2.at[pl.ds(sub * NBATCH, NBATCH)], flatv)
        pltpu.sync_copy(et2.at[pl.ds(sub * NBATCH, NBATCH)], etv)

        def _flat_body(j, carry):
            for k in range(BATCH // 16):
                sl = pl.ds(k * 16, 16)
                flatv[j, sl] = etv[j, sl] * N + flatv[j, sl]
            return carry
        lax.fori_loop(0, NBATCH, _flat_body, 0)

        def _ones(j, carry):
            onesv[j, pl.ds(0, 16)] = jnp.full((16,), 1.0, jnp.float32)
            z16[j, pl.ds(0, 16)] = jnp.zeros((16,), jnp.float32)
            return carry
        lax.fori_loop(0, BATCH, _ones, 0)

        def _zd(t, carry):
            pltpu.sync_copy(
                z16, deg_slab.at[pl.ds(sub * STRIPE + t * BATCH, BATCH)])
            return carry
        lax.fori_loop(0, STRIPE // BATCH, _zd, 0)
        plsc.subcore_barrier()

        def _dedge(b, carry):
            pltpu.sync_copy(onesv, deg_slab.at[flatv.at[b]], add=True)
            return carry
        lax.fori_loop(0, NBATCH, _dedge, 0)
        plsc.subcore_barrier()
        pltpu.sync_copy(deg_slab.at[pl.ds(sub * STRIPE, STRIPE)],
                        deg_out.at[pl.ds(sub * STRIPE, STRIPE)])


def _sc_agg(table, src2, dst2, et2):
    mesh = plsc.VectorSubcoreMesh(core_axis_name="c", subcore_axis_name="s")
    kern = pl.kernel(
        _sc_body,
        out_type=jax.ShapeDtypeStruct((SLAB_ROWS, H_DIM), jnp.float32),
        mesh=mesh,
        scratch_types=(
            pltpu.VMEM_SHARED((SLAB_ROWS, CW), jnp.float32),
            pltpu.VMEM((NBATCH, BATCH), jnp.int32),
            pltpu.VMEM((NBATCH, BATCH), jnp.int32),
            pltpu.VMEM((NBATCH, BATCH), jnp.int32),
            pltpu.VMEM((BATCH, CW), jnp.float32),
            pltpu.VMEM((BATCH, CW), jnp.float32),
        ),
        compiler_params=pltpu.CompilerParams(use_tc_tiling_on_sc=False),
    )
    return kern(table, src2, dst2, et2)


def _sc_deg(dst2, et2):
    mesh = plsc.VectorSubcoreMesh(core_axis_name="c", subcore_axis_name="s")
    kern = pl.kernel(
        _sc_deg_body,
        out_type=jax.ShapeDtypeStruct((SLAB_ROWS, 16), jnp.float32),
        mesh=mesh,
        scratch_types=(
            pltpu.VMEM_SHARED((SLAB_ROWS, 16), jnp.float32),
            pltpu.VMEM((NBATCH, BATCH), jnp.int32),
            pltpu.VMEM((NBATCH, BATCH), jnp.int32),
            pltpu.VMEM((BATCH, 16), jnp.float32),
            pltpu.VMEM((BATCH, 16), jnp.float32),
        ),
        compiler_params=pltpu.CompilerParams(use_tc_tiling_on_sc=False),
    )
    return kern(dst2, et2)


M_BLK = 2000
NB_N = 5
NBLK = N // NB_N


def _proj_kernel(x_ref, w_ref, b_ref, o_ref):
    o_ref[...] = (jnp.dot(x_ref[...], w_ref[...],
                          preferred_element_type=jnp.float32) + b_ref[...])


def _project(x, w, b):
    return pl.pallas_call(
        _proj_kernel,
        grid=(N // M_BLK,),
        in_specs=[pl.BlockSpec((M_BLK, IN_DIM), lambda i: (i, 0)),
                  pl.BlockSpec((IN_DIM, H_DIM), lambda i: (0, 0)),
                  pl.BlockSpec((1, H_DIM), lambda i: (0, 0))],
        out_specs=pl.BlockSpec((M_BLK, H_DIM), lambda i: (i, 0)),
        out_shape=jax.ShapeDtypeStruct((N, H_DIM), jnp.float32),
    )(x, w, b.reshape(1, H_DIM))


def _comb0_kernel(agg_ref, deg_ref, b_ref, o_ref):
    r = pl.program_id(1)
    inv = 1.0 / jnp.maximum(deg_ref[:, 0:1], 1.0)
    contrib = agg_ref[...] * inv

    @pl.when(r == 0)
    def _init():
        o_ref[...] = contrib

    @pl.when(r != 0)
    def _acc():
        o_ref[...] = o_ref[...] + contrib

    @pl.when(r == R - 1)
    def _fin():
        o_ref[...] = jnp.maximum(o_ref[...] + b_ref[...], 0.0)


def _combine0(agg, deg, bias):
    return pl.pallas_call(
        _comb0_kernel,
        grid=(NB_N, R),
        in_specs=[
            pl.BlockSpec((NBLK, H_DIM), lambda nb, r: (r * NB_N + nb, 0)),
            pl.BlockSpec((NBLK, 16), lambda nb, r: (r * NB_N + nb, 0)),
            pl.BlockSpec((1, H_DIM), lambda nb, r: (0, 0)),
        ],
        out_specs=pl.BlockSpec((NBLK, H_DIM), lambda nb, r: (nb, 0)),
        out_shape=jax.ShapeDtypeStruct((N, H_DIM), jnp.float32),
    )(agg, deg, bias.reshape(1, H_DIM))


def _combz_kernel(agg_ref, deg_ref, comp_ref, o_ref):
    r = pl.program_id(1)
    inv = 1.0 / jnp.maximum(deg_ref[:, 0:1], 1.0)
    base = agg_ref[...] * inv
    contrib = jnp.stack([comp_ref[r, 0] * base, comp_ref[r, 1] * base], axis=0)

    @pl.when(r == 0)
    def _init():
        o_ref[...] = contrib

    @pl.when(r != 0)
    def _acc():
        o_ref[...] = o_ref[...] + contrib


def _combinez(agg, deg, comp):
    return pl.pallas_call(
        _combz_kernel,
        grid=(NB_N, R),
        in_specs=[
            pl.BlockSpec((NBLK, H_DIM), lambda nb, r: (r * NB_N + nb, 0)),
            pl.BlockSpec((NBLK, 16), lambda nb, r: (r * NB_N + nb, 0)),
            pl.BlockSpec((R, NUM_B), lambda nb, r: (0, 0),
                         memory_space=pltpu.SMEM),
        ],
        out_specs=pl.BlockSpec((NUM_B, NBLK, H_DIM), lambda nb, r: (0, nb, 0)),
        out_shape=jax.ShapeDtypeStruct((NUM_B, N, H_DIM), jnp.float32),
    )(agg, deg, comp)


def _mm_kernel(relu, z_ref, w_ref, b_ref, o_ref):
    acc = jnp.dot(z_ref[0], w_ref[0], preferred_element_type=jnp.float32)
    acc = acc + jnp.dot(z_ref[1], w_ref[1], preferred_element_type=jnp.float32)
    acc = acc + b_ref[...]
    o_ref[...] = jnp.maximum(acc, 0.0) if relu else acc


def _basis_mm(z, basis, bias, relu):
    dout = basis.shape[-1]
    return pl.pallas_call(
        functools.partial(_mm_kernel, relu),
        grid=(N // M_BLK,),
        in_specs=[
            pl.BlockSpec((NUM_B, M_BLK, H_DIM), lambda i: (0, i, 0)),
            pl.BlockSpec((NUM_B, H_DIM, dout), lambda i: (0, 0, 0)),
            pl.BlockSpec((1, dout), lambda i: (0, 0)),
        ],
        out_specs=pl.BlockSpec((M_BLK, dout), lambda i: (i, 0)),
        out_shape=jax.ShapeDtypeStruct((N, dout), jnp.float32),
    )(z, basis, bias.reshape(1, dout))


def kernel(x, lin_W, lin_b, bias0, comp1, basis1, bias1, comp2, basis2, bias2,
           edge_index, edge_type):
    src = edge_index[0].astype(jnp.int32)
    dst = edge_index[1].astype(jnp.int32)
    et = edge_type.astype(jnp.int32)
    pad = E_PAD - E
    src2 = jnp.pad(src, (0, pad)).reshape(E_PAD // BATCH, BATCH)
    dst2 = jnp.pad(dst, (0, pad), constant_values=DUMMY_ROW).reshape(
        E_PAD // BATCH, BATCH)
    et2 = jnp.pad(et, (0, pad)).reshape(E_PAD // BATCH, BATCH)

    h0 = _project(x, lin_W, lin_b)
    deg = _sc_deg(dst2, et2)
    agg0 = _sc_agg(h0.reshape(N * NCHUNK, CW), src2, dst2, et2)
    h1 = _combine0(agg0, deg, bias0)

    agg1 = _sc_agg(h1.reshape(N * NCHUNK, CW), src2, dst2, et2)
    z1 = _combinez(agg1, deg, comp1)
    h2 = _basis_mm(z1, basis1, bias1, True)

    agg2 = _sc_agg(h2.reshape(N * NCHUNK, CW), src2, dst2, et2)
    z2 = _combinez(agg2, deg, comp2)
    return _basis_mm(z2, basis2, bias2, False)


# speedup vs baseline: 6.3160x; 1.3011x over previous
import functools

import jax
import jax.numpy as jnp
from jax import lax
from jax.experimental import pallas as pl
from jax.experimental.pallas import tpu as pltpu
from jax.experimental.pallas import tpu_sc as plsc

N = 10000
E = 160000
R = 4
IN_DIM = 128
H_DIM = 256
OUT_DIM = 64
NUM_B = 2

CW = 32
NBUF = 2
NCHUNK = H_DIM // CW
CORES = 2
TILES = 16
RN = R * N
SLAB_ROWS = 40960
STRIPE = SLAB_ROWS // TILES
DUMMY_ROW = RN
BATCH = 128
EPT = 10240
NBATCH = EPT // BATCH
E_PAD = EPT * TILES


def _sc_body(table, src2, dst2, et2, agg_out, slab, srcv, flatv, adjv,
             rowbuf, zbuf, gsem, ssem, zsem):
    core = lax.axis_index("c")
    sub = lax.axis_index("s")

    pltpu.sync_copy(src2.at[pl.ds(sub * NBATCH, NBATCH)], srcv)
    pltpu.sync_copy(dst2.at[pl.ds(sub * NBATCH, NBATCH)], flatv)
    pltpu.sync_copy(et2.at[pl.ds(sub * NBATCH, NBATCH)], adjv)

    def _flat_body(j, carry):
        for k in range(BATCH // 16):
            sl = pl.ds(k * 16, 16)
            flatv[j, sl] = adjv[j, sl] * N + flatv[j, sl]
        return carry
    lax.fori_loop(0, NBATCH, _flat_body, 0)

    def _zb(j, carry):
        z = jnp.zeros((16,), jnp.float32)
        for k in range(CW // 16):
            zbuf[j, pl.ds(k * 16, 16)] = z
        return carry
    lax.fori_loop(0, BATCH, _zb, 0)

    nzero = STRIPE // BATCH
    for lc in range(NCHUNK // CORES):
        chunk = core * (NCHUNK // CORES) + lc

        for t in range(nzero):
            pltpu.async_copy(
                zbuf, slab.at[pl.ds(sub * STRIPE + t * BATCH, BATCH)], zsem)

        def _adj(j, carry):
            for k in range(BATCH // 16):
                sl = pl.ds(k * 16, 16)
                adjv[j, sl] = srcv[j, sl] * NCHUNK + chunk
            return carry
        lax.fori_loop(0, NBATCH, _adj, 0)

        for t in range(nzero):
            pltpu.make_async_copy(
                zbuf, slab.at[pl.ds(sub * STRIPE + t * BATCH, BATCH)],
                zsem).wait()
        plsc.subcore_barrier()

        for k in range(NBUF):
            pltpu.async_copy(table.at[adjv.at[k]], rowbuf.at[k], gsem.at[k])

        def _grp(g, carry):
            for k in range(NBUF):
                b = g * NBUF + k
                pltpu.make_async_copy(
                    table.at[adjv.at[b]], rowbuf.at[k], gsem.at[k]).wait()
                pltpu.async_copy(
                    rowbuf.at[k], slab.at[flatv.at[b]], ssem.at[k], add=True)
                pltpu.make_async_copy(
                    rowbuf.at[k], slab.at[flatv.at[b]], ssem.at[k]).wait()

                @pl.when(b + NBUF < NBATCH)
                def _more():
                    pltpu.async_copy(table.at[adjv.at[b + NBUF]],
                                     rowbuf.at[k], gsem.at[k])
            return carry
        lax.fori_loop(0, NBATCH // NBUF, _grp, 0)
        plsc.subcore_barrier()

        pltpu.sync_copy(
            slab.at[pl.ds(sub * STRIPE, STRIPE)],
            agg_out.at[pl.ds(sub * STRIPE, STRIPE), pl.ds(chunk * CW, CW)])


def _sc_deg_body(dst2, et2, deg_out, deg_slab, flatv, etv, onesv, z16):
    core = lax.axis_index("c")
    sub = lax.axis_index("s")

    @pl.when(core == 0)
    def _deg_pass():
        pltpu.sync_copy(dst2.at[pl.ds(sub * NBATCH, NBATCH)], flatv)
        pltpu.sync_copy(et2.at[pl.ds(sub * NBATCH, NBATCH)], etv)

        def _flat_body(j, carry):
            for k in range(BATCH // 16):
                sl = pl.ds(k * 16, 16)
                flatv[j, sl] = etv[j, sl] * N + flatv[j, sl]
            return carry
        lax.fori_loop(0, NBATCH, _flat_body, 0)

        def _ones(j, carry):
            onesv[j, pl.ds(0, 16)] = jnp.full((16,), 1.0, jnp.float32)
            z16[j, pl.ds(0, 16)] = jnp.zeros((16,), jnp.float32)
            return carry
        lax.fori_loop(0, BATCH, _ones, 0)

        def _zd(t, carry):
            pltpu.sync_copy(
                z16, deg_slab.at[pl.ds(sub * STRIPE + t * BATCH, BATCH)])
            return carry
        lax.fori_loop(0, STRIPE // BATCH, _zd, 0)
        plsc.subcore_barrier()

        def _dedge(b, carry):
            pltpu.sync_copy(onesv, deg_slab.at[flatv.at[b]], add=True)
            return carry
        lax.fori_loop(0, NBATCH, _dedge, 0)
        plsc.subcore_barrier()
        pltpu.sync_copy(deg_slab.at[pl.ds(sub * STRIPE, STRIPE)],
                        deg_out.at[pl.ds(sub * STRIPE, STRIPE)])


def _sc_agg(table, src2, dst2, et2):
    mesh = plsc.VectorSubcoreMesh(core_axis_name="c", subcore_axis_name="s")
    kern = pl.kernel(
        _sc_body,
        out_type=jax.ShapeDtypeStruct((SLAB_ROWS, H_DIM), jnp.float32),
        mesh=mesh,
        scratch_types=(
            pltpu.VMEM_SHARED((SLAB_ROWS, CW), jnp.float32),
            pltpu.VMEM((NBATCH, BATCH), jnp.int32),
            pltpu.VMEM((NBATCH, BATCH), jnp.int32),
            pltpu.VMEM((NBATCH, BATCH), jnp.int32),
            pltpu.VMEM((NBUF, BATCH, CW), jnp.float32),
            pltpu.VMEM((BATCH, CW), jnp.float32),
            pltpu.SemaphoreType.DMA((NBUF,)),
            pltpu.SemaphoreType.DMA((NBUF,)),
            pltpu.SemaphoreType.DMA,
        ),
        compiler_params=pltpu.CompilerParams(use_tc_tiling_on_sc=False),
    )
    return kern(table, src2, dst2, et2)


def _sc_deg(dst2, et2):
    mesh = plsc.VectorSubcoreMesh(core_axis_name="c", subcore_axis_name="s")
    kern = pl.kernel(
        _sc_deg_body,
        out_type=jax.ShapeDtypeStruct((SLAB_ROWS, 16), jnp.float32),
        mesh=mesh,
        scratch_types=(
            pltpu.VMEM_SHARED((SLAB_ROWS, 16), jnp.float32),
            pltpu.VMEM((NBATCH, BATCH), jnp.int32),
            pltpu.VMEM((NBATCH, BATCH), jnp.int32),
            pltpu.VMEM((BATCH, 16), jnp.float32),
            pltpu.VMEM((BATCH, 16), jnp.float32),
        ),
        compiler_params=pltpu.CompilerParams(use_tc_tiling_on_sc=False),
    )
    return kern(dst2, et2)


M_BLK = 2000
NB_N = 5
NBLK = N // NB_N


def _proj_kernel(x_ref, w_ref, b_ref, o_ref):
    o_ref[...] = (jnp.dot(x_ref[...], w_ref[...],
                          preferred_element_type=jnp.float32) + b_ref[...])


def _project(x, w, b):
    return pl.pallas_call(
        _proj_kernel,
        grid=(N // M_BLK,),
        in_specs=[pl.BlockSpec((M_BLK, IN_DIM), lambda i: (i, 0)),
                  pl.BlockSpec((IN_DIM, H_DIM), lambda i: (0, 0)),
                  pl.BlockSpec((1, H_DIM), lambda i: (0, 0))],
        out_specs=pl.BlockSpec((M_BLK, H_DIM), lambda i: (i, 0)),
        out_shape=jax.ShapeDtypeStruct((N, H_DIM), jnp.float32),
    )(x, w, b.reshape(1, H_DIM))


def _comb0_kernel(agg_ref, deg_ref, b_ref, o_ref):
    r = pl.program_id(1)
    inv = 1.0 / jnp.maximum(deg_ref[:, 0:1], 1.0)
    contrib = agg_ref[...] * inv

    @pl.when(r == 0)
    def _init():
        o_ref[...] = contrib

    @pl.when(r != 0)
    def _acc():
        o_ref[...] = o_ref[...] + contrib

    @pl.when(r == R - 1)
    def _fin():
        o_ref[...] = jnp.maximum(o_ref[...] + b_ref[...], 0.0)


def _combine0(agg, deg, bias):
    return pl.pallas_call(
        _comb0_kernel,
        grid=(NB_N, R),
        in_specs=[
            pl.BlockSpec((NBLK, H_DIM), lambda nb, r: (r * NB_N + nb, 0)),
            pl.BlockSpec((NBLK, 16), lambda nb, r: (r * NB_N + nb, 0)),
            pl.BlockSpec((1, H_DIM), lambda nb, r: (0, 0)),
        ],
        out_specs=pl.BlockSpec((NBLK, H_DIM), lambda nb, r: (nb, 0)),
        out_shape=jax.ShapeDtypeStruct((N, H_DIM), jnp.float32),
    )(agg, deg, bias.reshape(1, H_DIM))


def _combz_kernel(agg_ref, deg_ref, comp_ref, o_ref):
    r = pl.program_id(1)
    inv = 1.0 / jnp.maximum(deg_ref[:, 0:1], 1.0)
    base = agg_ref[...] * inv
    contrib = jnp.stack([comp_ref[r, 0] * base, comp_ref[r, 1] * base], axis=0)

    @pl.when(r == 0)
    def _init():
        o_ref[...] = contrib

    @pl.when(r != 0)
    def _acc():
        o_ref[...] = o_ref[...] + contrib


def _combinez(agg, deg, comp):
    return pl.pallas_call(
        _combz_kernel,
        grid=(NB_N, R),
        in_specs=[
            pl.BlockSpec((NBLK, H_DIM), lambda nb, r: (r * NB_N + nb, 0)),
            pl.BlockSpec((NBLK, 16), lambda nb, r: (r * NB_N + nb, 0)),
            pl.BlockSpec((R, NUM_B), lambda nb, r: (0, 0),
                         memory_space=pltpu.SMEM),
        ],
        out_specs=pl.BlockSpec((NUM_B, NBLK, H_DIM), lambda nb, r: (0, nb, 0)),
        out_shape=jax.ShapeDtypeStruct((NUM_B, N, H_DIM), jnp.float32),
    )(agg, deg, comp)


def _mm_kernel(relu, z_ref, w_ref, b_ref, o_ref):
    acc = jnp.dot(z_ref[0], w_ref[0], preferred_element_type=jnp.float32)
    acc = acc + jnp.dot(z_ref[1], w_ref[1], preferred_element_type=jnp.float32)
    acc = acc + b_ref[...]
    o_ref[...] = jnp.maximum(acc, 0.0) if relu else acc


def _basis_mm(z, basis, bias, relu):
    dout = basis.shape[-1]
    return pl.pallas_call(
        functools.partial(_mm_kernel, relu),
        grid=(N // M_BLK,),
        in_specs=[
            pl.BlockSpec((NUM_B, M_BLK, H_DIM), lambda i: (0, i, 0)),
            pl.BlockSpec((NUM_B, H_DIM, dout), lambda i: (0, 0, 0)),
            pl.BlockSpec((1, dout), lambda i: (0, 0)),
        ],
        out_specs=pl.BlockSpec((M_BLK, dout), lambda i: (i, 0)),
        out_shape=jax.ShapeDtypeStruct((N, dout), jnp.float32),
    )(z, basis, bias.reshape(1, dout))


def kernel(x, lin_W, lin_b, bias0, comp1, basis1, bias1, comp2, basis2, bias2,
           edge_index, edge_type):
    src = edge_index[0].astype(jnp.int32)
    dst = edge_index[1].astype(jnp.int32)
    et = edge_type.astype(jnp.int32)
    pad = E_PAD - E
    src2 = jnp.pad(src, (0, pad)).reshape(E_PAD // BATCH, BATCH)
    dst2 = jnp.pad(dst, (0, pad), constant_values=DUMMY_ROW).reshape(
        E_PAD // BATCH, BATCH)
    et2 = jnp.pad(et, (0, pad)).reshape(E_PAD // BATCH, BATCH)

    h0 = _project(x, lin_W, lin_b)
    deg = _sc_deg(dst2, et2)
    agg0 = _sc_agg(h0.reshape(N * NCHUNK, CW), src2, dst2, et2)
    h1 = _combine0(agg0, deg, bias0)

    agg1 = _sc_agg(h1.reshape(N * NCHUNK, CW), src2, dst2, et2)
    z1 = _combinez(agg1, deg, comp1)
    h2 = _basis_mm(z1, basis1, bias1, True)

    agg2 = _sc_agg(h2.reshape(N * NCHUNK, CW), src2, dst2, et2)
    z2 = _combinez(agg2, deg, comp2)
    return _basis_mm(z2, basis2, bias2, False)


# speedup vs baseline: 7.0220x; 1.1118x over previous
import functools

import jax
import jax.numpy as jnp
from jax import lax
from jax.experimental import pallas as pl
from jax.experimental.pallas import tpu as pltpu
from jax.experimental.pallas import tpu_sc as plsc

N = 10000
E = 160000
R = 4
IN_DIM = 128
H_DIM = 256
OUT_DIM = 64
NUM_B = 2

CW = 32
NBUF = 5
NCHUNK = H_DIM // CW
CORES = 2
TILES = 16
RN = R * N
SLAB_ROWS = 40960
STRIPE = SLAB_ROWS // TILES
DUMMY_ROW = RN
BATCH = 128
EPT = 10240
NBATCH = EPT // BATCH
E_PAD = EPT * TILES


def _sc_body(table, src2, dst2, et2, agg_out, slab, flatv, adjv,
             rowbuf, zbuf, gsem, ssem, zsem):
    core = lax.axis_index("c")
    sub = lax.axis_index("s")

    pltpu.sync_copy(dst2.at[pl.ds(sub * NBATCH, NBATCH)], flatv)
    pltpu.sync_copy(et2.at[pl.ds(sub * NBATCH, NBATCH)], adjv)

    def _flat_body(j, carry):
        for k in range(BATCH // 16):
            sl = pl.ds(k * 16, 16)
            flatv[j, sl] = adjv[j, sl] * N + flatv[j, sl]
        return carry
    lax.fori_loop(0, NBATCH, _flat_body, 0)

    pltpu.sync_copy(src2.at[pl.ds(sub * NBATCH, NBATCH)], adjv)

    def _adj0(j, carry):
        for k in range(BATCH // 16):
            sl = pl.ds(k * 16, 16)
            adjv[j, sl] = adjv[j, sl] * NCHUNK + core * (NCHUNK // CORES)
        return carry
    lax.fori_loop(0, NBATCH, _adj0, 0)

    def _zb(j, carry):
        z = jnp.zeros((16,), jnp.float32)
        for k in range(CW // 16):
            zbuf[j, pl.ds(k * 16, 16)] = z
        return carry
    lax.fori_loop(0, BATCH, _zb, 0)

    nzero = STRIPE // BATCH
    for lc in range(NCHUNK // CORES):
        chunk = core * (NCHUNK // CORES) + lc

        for t in range(nzero):
            pltpu.async_copy(
                zbuf, slab.at[pl.ds(sub * STRIPE + t * BATCH, BATCH)], zsem)

        if lc > 0:
            def _adj(j, carry):
                for k in range(BATCH // 16):
                    sl = pl.ds(k * 16, 16)
                    adjv[j, sl] = adjv[j, sl] + 1
                return carry
            lax.fori_loop(0, NBATCH, _adj, 0)

        for t in range(nzero):
            pltpu.make_async_copy(
                zbuf, slab.at[pl.ds(sub * STRIPE + t * BATCH, BATCH)],
                zsem).wait()
        plsc.subcore_barrier()

        for k in range(NBUF):
            pltpu.async_copy(table.at[adjv.at[k]], rowbuf.at[k], gsem.at[k])

        def _grp(g, carry):
            for k in range(NBUF):
                b = g * NBUF + k
                pltpu.make_async_copy(
                    table.at[adjv.at[b]], rowbuf.at[k], gsem.at[k]).wait()
                pltpu.async_copy(
                    rowbuf.at[k], slab.at[flatv.at[b]], ssem.at[k], add=True)
                pltpu.make_async_copy(
                    rowbuf.at[k], slab.at[flatv.at[b]], ssem.at[k]).wait()

                @pl.when(b + NBUF < NBATCH)
                def _more():
                    pltpu.async_copy(table.at[adjv.at[b + NBUF]],
                                     rowbuf.at[k], gsem.at[k])
            return carry
        lax.fori_loop(0, NBATCH // NBUF, _grp, 0)
        plsc.subcore_barrier()

        pltpu.sync_copy(
            slab.at[pl.ds(sub * STRIPE, STRIPE)],
            agg_out.at[pl.ds(sub * STRIPE, STRIPE), pl.ds(chunk * CW, CW)])


def _sc_deg_body(dst2, et2, deg_out, deg_slab, flatv, etv, onesv, z16):
    core = lax.axis_index("c")
    sub = lax.axis_index("s")

    @pl.when(core == 0)
    def _deg_pass():
        pltpu.sync_copy(dst2.at[pl.ds(sub * NBATCH, NBATCH)], flatv)
        pltpu.sync_copy(et2.at[pl.ds(sub * NBATCH, NBATCH)], etv)

        def _flat_body(j, carry):
            for k in range(BATCH // 16):
                sl = pl.ds(k * 16, 16)
                flatv[j, sl] = etv[j, sl] * N + flatv[j, sl]
            return carry
        lax.fori_loop(0, NBATCH, _flat_body, 0)

        def _ones(j, carry):
            onesv[j, pl.ds(0, 16)] = jnp.full((16,), 1.0, jnp.float32)
            z16[j, pl.ds(0, 16)] = jnp.zeros((16,), jnp.float32)
            return carry
        lax.fori_loop(0, BATCH, _ones, 0)

        def _zd(t, carry):
            pltpu.sync_copy(
                z16, deg_slab.at[pl.ds(sub * STRIPE + t * BATCH, BATCH)])
            return carry
        lax.fori_loop(0, STRIPE // BATCH, _zd, 0)
        plsc.subcore_barrier()

        def _dedge(b, carry):
            pltpu.sync_copy(onesv, deg_slab.at[flatv.at[b]], add=True)
            return carry
        lax.fori_loop(0, NBATCH, _dedge, 0)
        plsc.subcore_barrier()
        pltpu.sync_copy(deg_slab.at[pl.ds(sub * STRIPE, STRIPE)],
                        deg_out.at[pl.ds(sub * STRIPE, STRIPE)])


def _sc_agg(table, src2, dst2, et2):
    mesh = plsc.VectorSubcoreMesh(core_axis_name="c", subcore_axis_name="s")
    kern = pl.kernel(
        _sc_body,
        out_type=jax.ShapeDtypeStruct((SLAB_ROWS, H_DIM), jnp.float32),
        mesh=mesh,
        scratch_types=(
            pltpu.VMEM_SHARED((SLAB_ROWS, CW), jnp.float32),
            pltpu.VMEM((NBATCH, BATCH), jnp.int32),
            pltpu.VMEM((NBATCH, BATCH), jnp.int32),
            pltpu.VMEM((NBUF, BATCH, CW), jnp.float32),
            pltpu.VMEM((BATCH, CW), jnp.float32),
            pltpu.SemaphoreType.DMA((NBUF,)),
            pltpu.SemaphoreType.DMA((NBUF,)),
            pltpu.SemaphoreType.DMA,
        ),
        compiler_params=pltpu.CompilerParams(use_tc_tiling_on_sc=False),
    )
    return kern(table, src2, dst2, et2)


def _sc_deg(dst2, et2):
    mesh = plsc.VectorSubcoreMesh(core_axis_name="c", subcore_axis_name="s")
    kern = pl.kernel(
        _sc_deg_body,
        out_type=jax.ShapeDtypeStruct((SLAB_ROWS, 16), jnp.float32),
        mesh=mesh,
        scratch_types=(
            pltpu.VMEM_SHARED((SLAB_ROWS, 16), jnp.float32),
            pltpu.VMEM((NBATCH, BATCH), jnp.int32),
            pltpu.VMEM((NBATCH, BATCH), jnp.int32),
            pltpu.VMEM((BATCH, 16), jnp.float32),
            pltpu.VMEM((BATCH, 16), jnp.float32),
        ),
        compiler_params=pltpu.CompilerParams(use_tc_tiling_on_sc=False),
    )
    return kern(dst2, et2)


M_BLK = 2000
NB_N = 5
NBLK = N // NB_N


def _proj_kernel(x_ref, w_ref, b_ref, o_ref):
    o_ref[...] = (jnp.dot(x_ref[...], w_ref[...],
                          preferred_element_type=jnp.float32) + b_ref[...])


def _project(x, w, b):
    return pl.pallas_call(
        _proj_kernel,
        grid=(N // M_BLK,),
        in_specs=[pl.BlockSpec((M_BLK, IN_DIM), lambda i: (i, 0)),
                  pl.BlockSpec((IN_DIM, H_DIM), lambda i: (0, 0)),
                  pl.BlockSpec((1, H_DIM), lambda i: (0, 0))],
        out_specs=pl.BlockSpec((M_BLK, H_DIM), lambda i: (i, 0)),
        out_shape=jax.ShapeDtypeStruct((N, H_DIM), jnp.float32),
    )(x, w, b.reshape(1, H_DIM))


def _comb0_kernel(agg_ref, deg_ref, b_ref, o_ref):
    r = pl.program_id(1)
    inv = 1.0 / jnp.maximum(deg_ref[:, 0:1], 1.0)
    contrib = agg_ref[...] * inv

    @pl.when(r == 0)
    def _init():
        o_ref[...] = contrib

    @pl.when(r != 0)
    def _acc():
        o_ref[...] = o_ref[...] + contrib

    @pl.when(r == R - 1)
    def _fin():
        o_ref[...] = jnp.maximum(o_ref[...] + b_ref[...], 0.0)


def _combine0(agg, deg, bias):
    return pl.pallas_call(
        _comb0_kernel,
        grid=(NB_N, R),
        in_specs=[
            pl.BlockSpec((NBLK, H_DIM), lambda nb, r: (r * NB_N + nb, 0)),
            pl.BlockSpec((NBLK, 16), lambda nb, r: (r * NB_N + nb, 0)),
            pl.BlockSpec((1, H_DIM), lambda nb, r: (0, 0)),
        ],
        out_specs=pl.BlockSpec((NBLK, H_DIM), lambda nb, r: (nb, 0)),
        out_shape=jax.ShapeDtypeStruct((N, H_DIM), jnp.float32),
    )(agg, deg, bias.reshape(1, H_DIM))


def _combz_kernel(agg_ref, deg_ref, comp_ref, o_ref):
    r = pl.program_id(1)
    inv = 1.0 / jnp.maximum(deg_ref[:, 0:1], 1.0)
    base = agg_ref[...] * inv
    contrib = jnp.stack([comp_ref[r, 0] * base, comp_ref[r, 1] * base], axis=0)

    @pl.when(r == 0)
    def _init():
        o_ref[...] = contrib

    @pl.when(r != 0)
    def _acc():
        o_ref[...] = o_ref[...] + contrib


def _combinez(agg, deg, comp):
    return pl.pallas_call(
        _combz_kernel,
        grid=(NB_N, R),
        in_specs=[
            pl.BlockSpec((NBLK, H_DIM), lambda nb, r: (r * NB_N + nb, 0)),
            pl.BlockSpec((NBLK, 16), lambda nb, r: (r * NB_N + nb, 0)),
            pl.BlockSpec((R, NUM_B), lambda nb, r: (0, 0),
                         memory_space=pltpu.SMEM),
        ],
        out_specs=pl.BlockSpec((NUM_B, NBLK, H_DIM), lambda nb, r: (0, nb, 0)),
        out_shape=jax.ShapeDtypeStruct((NUM_B, N, H_DIM), jnp.float32),
    )(agg, deg, comp)


def _mm_kernel(relu, z_ref, w_ref, b_ref, o_ref):
    acc = jnp.dot(z_ref[0], w_ref[0], preferred_element_type=jnp.float32)
    acc = acc + jnp.dot(z_ref[1], w_ref[1], preferred_element_type=jnp.float32)
    acc = acc + b_ref[...]
    o_ref[...] = jnp.maximum(acc, 0.0) if relu else acc


def _basis_mm(z, basis, bias, relu):
    dout = basis.shape[-1]
    return pl.pallas_call(
        functools.partial(_mm_kernel, relu),
        grid=(N // M_BLK,),
        in_specs=[
            pl.BlockSpec((NUM_B, M_BLK, H_DIM), lambda i: (0, i, 0)),
            pl.BlockSpec((NUM_B, H_DIM, dout), lambda i: (0, 0, 0)),
            pl.BlockSpec((1, dout), lambda i: (0, 0)),
        ],
        out_specs=pl.BlockSpec((M_BLK, dout), lambda i: (i, 0)),
        out_shape=jax.ShapeDtypeStruct((N, dout), jnp.float32),
    )(z, basis, bias.reshape(1, dout))


def kernel(x, lin_W, lin_b, bias0, comp1, basis1, bias1, comp2, basis2, bias2,
           edge_index, edge_type):
    src = edge_index[0].astype(jnp.int32)
    dst = edge_index[1].astype(jnp.int32)
    et = edge_type.astype(jnp.int32)
    pad = E_PAD - E
    src2 = jnp.pad(src, (0, pad)).reshape(E_PAD // BATCH, BATCH)
    dst2 = jnp.pad(dst, (0, pad), constant_values=DUMMY_ROW).reshape(
        E_PAD // BATCH, BATCH)
    et2 = jnp.pad(et, (0, pad)).reshape(E_PAD // BATCH, BATCH)

    h0 = _project(x, lin_W, lin_b)
    deg = _sc_deg(dst2, et2)
    agg0 = _sc_agg(h0.reshape(N * NCHUNK, CW), src2, dst2, et2)
    h1 = _combine0(agg0, deg, bias0)

    agg1 = _sc_agg(h1.reshape(N * NCHUNK, CW), src2, dst2, et2)
    z1 = _combinez(agg1, deg, comp1)
    h2 = _basis_mm(z1, basis1, bias1, True)

    agg2 = _sc_agg(h2.reshape(N * NCHUNK, CW), src2, dst2, et2)
    z2 = _combinez(agg2, deg, comp2)
    return _basis_mm(z2, basis2, bias2, False)
